# baseline (speedup 1.0000x reference)
import jax
import jax.numpy as jnp
from jax import lax
from jax.experimental import pallas as pl
from jax.experimental.pallas import tpu as pltpu
from jax.experimental.pallas import tpu_sc as plsc

_VOCAB = 100000
_EMBED_DIM = 64
_BATCH = 4096
_HIST = 50

_NW = 32
_BPW = _BATCH // _NW
_NQ = _HIST * _EMBED_DIM // 128
_NBUF = 10
_NGRP = _HIST // _NBUF


def _body(idxt_hbm, table_hbm, out_hbm, idx_v, rows_v, *sems):
    gsem = sems[:_NBUF]
    wsem = sems[_NBUF:]
    wid = lax.axis_index("s") * 2 + lax.axis_index("c")
    b0 = wid * _BPW
    pltpu.sync_copy(idxt_hbm.at[:, pl.ds(b0, _BPW)], idx_v)

    def gather(h, s):
        return pltpu.make_async_copy(
            table_hbm.at[idx_v.at[h]], rows_v.at[s], gsem[s])

    def write(h, s, parity):
        return pltpu.make_async_copy(
            rows_v.at[s],
            out_hbm.at[wid, (h - parity) // 2,
                       :, pl.ds(parity * _EMBED_DIM, _EMBED_DIM)],
            wsem[s])

    for s in range(_NBUF):
        gather(s, s).start()

    @pl.loop(0, _NGRP - 1)
    def grp(g):
        h0 = g * _NBUF
        for s in range(_NBUF):
            gather(h0 + s, s).wait()
            write(h0 + s, s, s % 2).start()
        for s in range(_NBUF):
            write(h0 + s, s, s % 2).wait()
            gather(h0 + _NBUF + s, s).start()

    h0 = (_NGRP - 1) * _NBUF
    for s in range(_NBUF):
        gather(h0 + s, s).wait()
        write(h0 + s, s, s % 2).start()
    for s in range(_NBUF):
        write(h0 + s, s, s % 2).wait()


@jax.jit
def _embed(idxt, table):
    mesh = plsc.VectorSubcoreMesh(core_axis_name="c", subcore_axis_name="s")
    f = pl.kernel(
        _body,
        out_type=jax.ShapeDtypeStruct((_NW, _NQ, _BPW, 128), jnp.float32),
        mesh=mesh,
        scratch_types=[
            pltpu.VMEM((_HIST, _BPW), jnp.int32),
            pltpu.VMEM((_NBUF, _BPW, _EMBED_DIM), jnp.float32),
        ] + [pltpu.SemaphoreType.DMA] * (2 * _NBUF),
        compiler_params=pltpu.CompilerParams(
            use_tc_tiling_on_sc=False, needs_layout_passes=False),
    )
    return f(idxt, table)


def _tc_transpose_body(x_ref, o_ref):
    o_ref[...] = x_ref[0, 0].T


def _tc_transpose(x4):
    return pl.pallas_call(
        _tc_transpose_body,
        grid=(_NW, _NQ),
        in_specs=[pl.BlockSpec((1, 1, _BPW, 128), lambda w, q: (w, q, 0, 0))],
        out_specs=pl.BlockSpec((128, _BPW), lambda w, q: (q, w)),
        out_shape=jax.ShapeDtypeStruct((_HIST * _EMBED_DIM, _BATCH),
                                       jnp.float32),
    )(x4)


def kernel(input, table):
    idxt = input.T.astype(jnp.int32)
    out1 = _embed(idxt, table)
    t2d = _tc_transpose(out1)
    return t2d.reshape(_HIST, _EMBED_DIM, _BATCH).transpose(2, 0, 1)


# speedup vs baseline: 3.3907x; 3.3907x over previous
import jax
import jax.numpy as jnp
from jax import lax
from jax.experimental import pallas as pl
from jax.experimental.pallas import tpu as pltpu
from jax.experimental.pallas import tpu_sc as plsc

_VOCAB = 100000
_EMBED_DIM = 64
_BATCH = 4096
_HIST = 50

_NW = 32
_BPW = _BATCH // _NW
_NQ = _HIST * _EMBED_DIM // 128
_NBUF = 10
_NGRP = _HIST // _NBUF


def _body(idxt_hbm, table_hbm, out_hbm, idx_v, rows_v, *sems):
    gsem = sems[:_NBUF]
    wsem = sems[_NBUF:]
    wid = lax.axis_index("s") * 2 + lax.axis_index("c")
    b0 = wid * _BPW
    pltpu.sync_copy(idxt_hbm.at[:, pl.ds(b0, _BPW)], idx_v)

    def gather(h, s):
        return pltpu.make_async_copy(
            table_hbm.at[idx_v.at[h]], rows_v.at[s], gsem[s])

    def write(h, s, parity):
        return pltpu.make_async_copy(
            rows_v.at[s],
            out_hbm.at[wid, (h - parity) // 2,
                       :, pl.ds(parity * _EMBED_DIM, _EMBED_DIM)],
            wsem[s])

    for s in range(_NBUF):
        gather(s, s).start()

    @pl.loop(0, _NGRP - 1)
    def grp(g):
        h0 = g * _NBUF
        for s in range(_NBUF):
            gather(h0 + s, s).wait()
            write(h0 + s, s, s % 2).start()
        for s in range(_NBUF):
            write(h0 + s, s, s % 2).wait()
            gather(h0 + _NBUF + s, s).start()

    h0 = (_NGRP - 1) * _NBUF
    for s in range(_NBUF):
        gather(h0 + s, s).wait()
        write(h0 + s, s, s % 2).start()
    for s in range(_NBUF):
        write(h0 + s, s, s % 2).wait()


@jax.jit
def _embed(idxt, table):
    mesh = plsc.VectorSubcoreMesh(core_axis_name="c", subcore_axis_name="s")
    f = pl.kernel(
        _body,
        out_type=jax.ShapeDtypeStruct((_NW, _NQ, _BPW, 128), jnp.float32),
        mesh=mesh,
        scratch_types=[
            pltpu.VMEM((_HIST, _BPW), jnp.int32),
            pltpu.VMEM((_NBUF, _BPW, _EMBED_DIM), jnp.float32),
        ] + [pltpu.SemaphoreType.DMA] * (2 * _NBUF),
        compiler_params=pltpu.CompilerParams(
            use_tc_tiling_on_sc=False, needs_layout_passes=False),
    )
    return f(idxt, table)


def _tc_transpose_body(x_ref, o_ref):
    eye = jnp.eye(128, dtype=jnp.float32)
    for q in range(_NQ):
        o_ref[pl.ds(q * 128, 128), :] = lax.dot_general(
            x_ref[0, q], eye, (((0,), (0,)), ((), ())),
            preferred_element_type=jnp.float32)


def _tc_transpose(x4):
    return pl.pallas_call(
        _tc_transpose_body,
        grid=(_NW,),
        in_specs=[pl.BlockSpec((1, _NQ, _BPW, 128), lambda w: (w, 0, 0, 0))],
        out_specs=pl.BlockSpec((_HIST * _EMBED_DIM, _BPW), lambda w: (0, w)),
        out_shape=jax.ShapeDtypeStruct((_HIST * _EMBED_DIM, _BATCH),
                                       jnp.float32),
    )(x4)


def kernel(input, table):
    idxt = input.T.astype(jnp.int32)
    out1 = _embed(idxt, table)
    t2d = _tc_transpose(out1)
    return t2d.reshape(_HIST, _EMBED_DIM, _BATCH).transpose(2, 0, 1)
